# Initial kernel scaffold; baseline (speedup 1.0000x reference)
import jax
import jax.numpy as jnp
from jax.experimental import pallas as pl


def kernel(x_paper, x_author, edge_index_writes, edge_index_written_by, params):
    raise NotImplementedError("write your pallas kernel here")


# speedup vs baseline: 2.0921x; 2.0921x over previous
import functools
import math

import jax
import jax.numpy as jnp
from jax import lax
from jax.experimental import pallas as pl
from jax.experimental.pallas import tpu as pltpu
from jax.experimental.pallas import tpu_sc as plsc

N = 50000
E = 300000
D = 128
H = 8
DK = 16
BN = 2000
C = 96
NCHUNK = E // C
NSUB = 16
ROWS_PER_TILE = N // NSUB
ZROWS = 625


def _proj_body(x_ref, w_ref, b_ref, q_ref, kt_ref, vt_ref):
    y = jnp.dot(x_ref[...], w_ref[...], preferred_element_type=jnp.float32)
    y = y + b_ref[...][None, :]
    q_ref[...] = y[:, 0:D]
    kt_ref[...] = y[:, D:2 * D]
    vt_ref[...] = y[:, 2 * D:3 * D]


def _proj(x, wcat, bcat):
    return pl.pallas_call(
        _proj_body,
        grid=(N // BN,),
        in_specs=[
            pl.BlockSpec((BN, D), lambda i: (i, 0)),
            pl.BlockSpec((D, 3 * D), lambda i: (0, 0)),
            pl.BlockSpec((3 * D,), lambda i: (0,)),
        ],
        out_specs=[
            pl.BlockSpec((BN, D), lambda i: (i, 0)),
            pl.BlockSpec((BN, D), lambda i: (i, 0)),
            pl.BlockSpec((BN, D), lambda i: (i, 0)),
        ],
        out_shape=[jax.ShapeDtypeStruct((N, D), jnp.float32)] * 3,
    )(x, wcat, bcat)


def _fin_body(acc_ref, den_ref, x_ref, wa_ref, ba_ref, skip_ref, o_ref):
    acc = acc_ref[...]
    den = den_ref[...]
    m = jnp.where(den > 0.0, acc / jnp.maximum(den, 1e-30), 0.0)
    m = jnp.maximum(m, 0.0)
    y = jnp.dot(m, wa_ref[...], preferred_element_type=jnp.float32)
    y = y + ba_ref[...][None, :]
    alpha = 1.0 / (1.0 + jnp.exp(-skip_ref[0, 0]))
    o_ref[...] = y * alpha + x_ref[...] * (1.0 - alpha)


def _finalize(acc, den, x, wa, ba, skip):
    return pl.pallas_call(
        _fin_body,
        grid=(N // BN,),
        in_specs=[
            pl.BlockSpec((BN, D), lambda i: (i, 0)),
            pl.BlockSpec((BN, D), lambda i: (i, 0)),
            pl.BlockSpec((BN, D), lambda i: (i, 0)),
            pl.BlockSpec((D, D), lambda i: (0, 0)),
            pl.BlockSpec((D,), lambda i: (0,)),
            pl.BlockSpec((1, 1), lambda i: (0, 0)),
        ],
        out_specs=pl.BlockSpec((BN, D), lambda i: (i, 0)),
        out_shape=jax.ShapeDtypeStruct((N, D), jnp.float32),
    )(acc, den, x, wa, ba, skip)


def _edge_pass(c, src_hbm, dst_hbm, kt_hbm, q_hbm, vt_hbm, acc_hbm, den_hbm,
               slab, zero_vm, src_vm, dst_vm, kidx_vm, qidx_vm,
               kt_st, q_st, vt_st, stage, gsem):
    sid = lax.axis_index("s")
    row0 = sid * ROWS_PER_TILE

    def head_body(i, _):
        h = 2 * i + c

        def clr(k, _):
            pltpu.sync_copy(zero_vm, slab.at[pl.ds(row0 + k * ZROWS, ZROWS)])
            return _
        lax.fori_loop(0, ROWS_PER_TILE // ZROWS, clr, 0)
        plsc.subcore_barrier()

        nj = (NCHUNK - sid + NSUB - 1) // NSUB

        def chunk(kk, _):
            base = (kk * NSUB + sid) * C
            pltpu.sync_copy(src_hbm.at[pl.ds(base, C)], src_vm)
            pltpu.sync_copy(dst_hbm.at[pl.ds(base, C)], dst_vm)

            def mkidx(g, _):
                sl = pl.ds(g * 16, 16)
                kidx_vm[sl] = src_vm[sl] * H + h
                qidx_vm[sl] = dst_vm[sl] * H + h
                return _
            lax.fori_loop(0, C // 16, mkidx, 0)

            cp1 = pltpu.async_copy(kt_hbm.at[kidx_vm], kt_st, gsem)
            cp2 = pltpu.async_copy(q_hbm.at[qidx_vm], q_st, gsem)
            cp3 = pltpu.async_copy(vt_hbm.at[kidx_vm], vt_st, gsem)
            cp1.wait()
            cp2.wait()
            cp3.wait()

            def edge(e, _):
                prod = kt_st[e] * q_st[e]
                s = jnp.sum(prod)
                ex = jnp.exp(jnp.broadcast_to(s, (16,)))
                stage[e, 0, :] = vt_st[e] * ex
                stage[e, 1, :] = ex
                return _
            lax.fori_loop(0, C, edge, 0)

            pltpu.sync_copy(stage, slab.at[dst_vm], add=True)
            return _
        lax.fori_loop(0, nj, chunk, 0)
        plsc.subcore_barrier()

        rows = pl.ds(row0, ROWS_PER_TILE)
        pltpu.sync_copy(slab.at[rows, 0, :], acc_hbm.at[rows, h, :])
        pltpu.sync_copy(slab.at[rows, 1, :], den_hbm.at[rows, h, :])
        plsc.subcore_barrier()
        return _
    lax.fori_loop(0, H // 2, head_body, 0)


def _sc_edges(src_w, dst_w, src_wb, dst_wb, ktw, qp, vtw, ktwb, qa, vtwb):
    mesh = plsc.VectorSubcoreMesh(core_axis_name="c", subcore_axis_name="s",
                                  num_cores=2, num_subcores=NSUB)

    @functools.partial(
        pl.kernel,
        out_type=[jax.ShapeDtypeStruct((N, H, DK), jnp.float32)] * 4,
        mesh=mesh,
        scratch_types=[
            pltpu.VMEM_SHARED((N, 2, DK), jnp.float32),
            pltpu.VMEM((ZROWS, 2, DK), jnp.float32),
            pltpu.VMEM((C,), jnp.int32),
            pltpu.VMEM((C,), jnp.int32),
            pltpu.VMEM((C,), jnp.int32),
            pltpu.VMEM((C,), jnp.int32),
            pltpu.VMEM((C, DK), jnp.float32),
            pltpu.VMEM((C, DK), jnp.float32),
            pltpu.VMEM((C, DK), jnp.float32),
            pltpu.VMEM((C, 2, DK), jnp.float32),
            pltpu.SemaphoreType.DMA,
        ],
    )
    def k(src_w_h, dst_w_h, src_wb_h, dst_wb_h, ktw_h, qp_h, vtw_h,
          ktwb_h, qa_h, vtwb_h, accp_h, denp_h, acca_h, dena_h,
          slab, zero_vm, src_vm, dst_vm, kidx_vm, qidx_vm,
          kt_st, q_st, vt_st, stage, gsem):
        c = lax.axis_index("c")

        z = jnp.zeros((16,), jnp.float32)

        def zinit(r, _):
            zero_vm[r, 0, :] = z
            zero_vm[r, 1, :] = z
            return _
        lax.fori_loop(0, ZROWS, zinit, 0)

        _edge_pass(c, src_w_h, dst_w_h, ktw_h, qp_h, vtw_h, accp_h, denp_h,
                   slab, zero_vm, src_vm, dst_vm, kidx_vm, qidx_vm,
                   kt_st, q_st, vt_st, stage, gsem)
        _edge_pass(c, src_wb_h, dst_wb_h, ktwb_h, qa_h, vtwb_h, acca_h, dena_h,
                   slab, zero_vm, src_vm, dst_vm, kidx_vm, qidx_vm,
                   kt_st, q_st, vt_st, stage, gsem)

    return k(src_w, dst_w, src_wb, dst_wb, ktw, qp, vtw, ktwb, qa, vtwb)


def _fold_weights(params):
    sqrt_dk = math.sqrt(DK)
    out = {}
    for e, srct, dstt in (("writes", "author", "paper"),
                          ("written_by", "paper", "author")):
        att = params["att_%s" % e] * (params["pri_%s" % e] / sqrt_dk)[:, None, None]
        wk = params["Wk_%s" % srct].reshape(D, H, DK)
        wkt = jnp.einsum("ihd,hdf->ihf", wk, att).reshape(D, D)
        bkt = jnp.einsum("hd,hdf->hf", params["bk_%s" % srct].reshape(H, DK),
                         att).reshape(-1)
        wv = params["Wv_%s" % srct].reshape(D, H, DK)
        wvt = jnp.einsum("ihd,hdf->ihf", wv, params["msg_%s" % e]).reshape(D, D)
        bvt = jnp.einsum("hd,hdf->hf", params["bv_%s" % srct].reshape(H, DK),
                         params["msg_%s" % e]).reshape(-1)
        out[e] = (wkt, bkt, wvt, bvt)
    return out


@jax.jit
def kernel(x_paper, x_author, edge_index_writes, edge_index_written_by, params):
    fw = _fold_weights(params)
    wkt_w, bkt_w, wvt_w, bvt_w = fw["writes"]
    wkt_wb, bkt_wb, wvt_wb, bvt_wb = fw["written_by"]

    wcat_p = jnp.concatenate([params["Wq_paper"], wkt_wb, wvt_wb], axis=1)
    bcat_p = jnp.concatenate([params["bq_paper"], bkt_wb, bvt_wb], axis=0)
    wcat_a = jnp.concatenate([params["Wq_author"], wkt_w, wvt_w], axis=1)
    bcat_a = jnp.concatenate([params["bq_author"], bkt_w, bvt_w], axis=0)

    q_p, kt_wb, vt_wb = _proj(x_paper, wcat_p, bcat_p)
    q_a, kt_w, vt_w = _proj(x_author, wcat_a, bcat_a)

    def edge_jnp(src, dst, kt, q, vt):
        score = (kt[src].reshape(-1, H, DK) * q[dst].reshape(-1, H, DK)).sum(-1)
        ex = jnp.exp(score)
        den = jax.ops.segment_sum(ex, dst, num_segments=N)
        acc = jax.ops.segment_sum(ex[:, :, None] * vt[src].reshape(-1, H, DK),
                                  dst, num_segments=N)
        return acc, jnp.repeat(den, DK, axis=1)

    accp, denp = edge_jnp(edge_index_writes[0], edge_index_writes[1],
                          kt_w, q_p, vt_w)
    acca, dena = edge_jnp(edge_index_written_by[0], edge_index_written_by[1],
                          kt_wb, q_a, vt_wb)

    out_p = _finalize(accp.reshape(N, D), denp.reshape(N, D), x_paper,
                      params["Wa_paper"], params["ba_paper"],
                      params["skip_paper"].reshape(1, 1))
    out_a = _finalize(acca.reshape(N, D), dena.reshape(N, D), x_author,
                      params["Wa_author"], params["ba_author"],
                      params["skip_author"].reshape(1, 1))
    return (out_p, out_a)


# speedup vs baseline: 4.2497x; 2.0313x over previous
import functools
import math

import jax
import jax.numpy as jnp
from jax import lax
from jax.experimental import pallas as pl
from jax.experimental.pallas import tpu as pltpu
from jax.experimental.pallas import tpu_sc as plsc

N = 50000
E = 300000
D = 128
H = 8
DK = 16
BN = 2000
NSUB = 16

NBK = 25
BROW = 2048
SLABR = 2056
DUMMY = BROW
CAP = 976
FILL = 960
RSTRIDE = NBK * CAP
PADE = 32 * 9600
NREG = 320
NPAD_OUT = NBK * BROW


def _proj_body(x_ref, w_ref, b_ref, q_ref, kt_ref, vt_ref):
    y = jnp.dot(x_ref[...], w_ref[...], preferred_element_type=jnp.float32)
    y = y + b_ref[...][None, :]
    q_ref[...] = y[:, 0:D]
    kt_ref[...] = y[:, D:2 * D]
    vt_ref[...] = y[:, 2 * D:3 * D]


def _proj(x, wcat, bcat):
    return pl.pallas_call(
        _proj_body,
        grid=(N // BN,),
        in_specs=[
            pl.BlockSpec((BN, D), lambda i: (i, 0)),
            pl.BlockSpec((D, 3 * D), lambda i: (0, 0)),
            pl.BlockSpec((3 * D,), lambda i: (0,)),
        ],
        out_specs=[
            pl.BlockSpec((BN, D), lambda i: (i, 0)),
            pl.BlockSpec((BN, D), lambda i: (i, 0)),
            pl.BlockSpec((BN, D), lambda i: (i, 0)),
        ],
        out_shape=[jax.ShapeDtypeStruct((N, D), jnp.float32)] * 3,
    )(x, wcat, bcat)


def _fin_body(acc_ref, den_ref, x_ref, wa_ref, ba_ref, skip_ref, o_ref):
    acc = acc_ref[...]
    den = den_ref[...]
    m = jnp.where(den > 0.0, acc / jnp.maximum(den, 1e-30), 0.0)
    m = jnp.maximum(m, 0.0)
    y = jnp.dot(m, wa_ref[...], preferred_element_type=jnp.float32)
    y = y + ba_ref[...][None, :]
    alpha = 1.0 / (1.0 + jnp.exp(-skip_ref[0, 0]))
    o_ref[...] = y * alpha + x_ref[...] * (1.0 - alpha)


def _finalize(acc, den, x, wa, ba, skip):
    return pl.pallas_call(
        _fin_body,
        grid=(N // BN,),
        in_specs=[
            pl.BlockSpec((BN, D), lambda i: (i, 0)),
            pl.BlockSpec((BN, D), lambda i: (i, 0)),
            pl.BlockSpec((BN, D), lambda i: (i, 0)),
            pl.BlockSpec((D, D), lambda i: (0, 0)),
            pl.BlockSpec((D,), lambda i: (0,)),
            pl.BlockSpec((1, 1), lambda i: (0, 0)),
        ],
        out_specs=pl.BlockSpec((BN, D), lambda i: (i, 0)),
        out_shape=jax.ShapeDtypeStruct((N, D), jnp.float32),
    )(acc, den, x, wa, ba, skip)


_MESH = None


def _mesh():
    global _MESH
    if _MESH is None:
        _MESH = plsc.VectorSubcoreMesh(core_axis_name="c", subcore_axis_name="s",
                                       num_cores=2, num_subcores=NSUB)
    return _MESH


def _prepass_etype(src_h, dst_h, bs_h, bd_h, cn_h, sv, dv, cv, sb, db, smem):
    sid = lax.axis_index("s")
    c = lax.axis_index("c")
    w = c * NSUB + sid
    iota = lax.iota(jnp.int32, 16)

    def tenth(tt, _):
        reg = w * 10 + tt

        def zc(i, _):
            smem[i] = 0
            return _
        lax.fori_loop(0, 26, zc, 0)

        def chunk(ck, _):
            base = w * 9600 + tt * FILL + ck * 96
            pltpu.sync_copy(src_h.at[pl.ds(base, 96)], sv)
            pltpu.sync_copy(dst_h.at[pl.ds(base, 96)], dv)

            def grp(g, _):
                sl = pl.ds(g * 16, 16)
                svec = sv[sl]
                dvec = dv[sl]
                for j in range(16):
                    dj = dvec[j]
                    qj = dj >> 11
                    cc = smem[qj]
                    pos = qj * CAP + cc
                    sb[pl.ds(pos, 16)] = jnp.broadcast_to(svec[j], (16,))
                    db[pl.ds(pos, 16)] = jnp.broadcast_to(dj, (16,))
                    smem[qj] = cc + 1
                return _
            lax.fori_loop(0, 6, grp, 0)
            return _
        lax.fori_loop(0, 10, chunk, 0)

        cv0 = jnp.zeros((16,), jnp.int32)
        cv1 = jnp.zeros((16,), jnp.int32)

        def b0(i, v):
            return jnp.where(iota == i, jnp.broadcast_to(smem[i], (16,)), v)

        def b1(i, v):
            return jnp.where(iota == i, jnp.broadcast_to(smem[i + 16], (16,)), v)
        cv0 = lax.fori_loop(0, 16, b0, cv0)
        cv1 = lax.fori_loop(0, 9, b1, cv1)
        cv[pl.ds(0, 16)] = cv0
        cv[pl.ds(16, 16)] = cv1
        pltpu.sync_copy(cv, cn_h.at[pl.ds(reg * 32, 32)])
        pltpu.sync_copy(sb.at[pl.ds(0, RSTRIDE)],
                        bs_h.at[pl.ds(reg * RSTRIDE, RSTRIDE)])
        pltpu.sync_copy(db.at[pl.ds(0, RSTRIDE)],
                        bd_h.at[pl.ds(reg * RSTRIDE, RSTRIDE)])
        return _
    lax.fori_loop(0, 10, tenth, 0)


def _sc_prepass(src_w, dst_w, src_wb, dst_wb):
    @functools.partial(
        pl.kernel,
        out_type=[jax.ShapeDtypeStruct((NREG * RSTRIDE,), jnp.int32),
                  jax.ShapeDtypeStruct((NREG * RSTRIDE,), jnp.int32),
                  jax.ShapeDtypeStruct((NREG * 32,), jnp.int32),
                  jax.ShapeDtypeStruct((NREG * RSTRIDE,), jnp.int32),
                  jax.ShapeDtypeStruct((NREG * RSTRIDE,), jnp.int32),
                  jax.ShapeDtypeStruct((NREG * 32,), jnp.int32)],
        mesh=_mesh(),
        scratch_types=[
            pltpu.VMEM((96,), jnp.int32),
            pltpu.VMEM((96,), jnp.int32),
            pltpu.VMEM((32,), jnp.int32),
            pltpu.VMEM((26 * CAP,), jnp.int32),
            pltpu.VMEM((26 * CAP,), jnp.int32),
            pltpu.SMEM((32,), jnp.int32),
        ],
    )
    def k(srcw_h, dstw_h, srcwb_h, dstwb_h,
          bsw_h, bdw_h, cnw_h, bswb_h, bdwb_h, cnwb_h,
          sv, dv, cv, sb, db, smem):
        _prepass_etype(srcw_h, dstw_h, bsw_h, bdw_h, cnw_h,
                       sv, dv, cv, sb, db, smem)
        _prepass_etype(srcwb_h, dstwb_h, bswb_h, bdwb_h, cnwb_h,
                       sv, dv, cv, sb, db, smem)

    return k(src_w, dst_w, src_wb, dst_wb)


def _sc_main(kt, qd, vt, bs, bd, cn, b_lo, nbk):
    @functools.partial(
        pl.kernel,
        out_type=[jax.ShapeDtypeStruct((nbk * BROW, D), jnp.float32),
                  jax.ShapeDtypeStruct((nbk * BROW, D), jnp.float32)],
        mesh=_mesh(),
        scratch_types=[
            pltpu.VMEM((96,), jnp.int32),
            pltpu.VMEM((96,), jnp.int32),
            pltpu.VMEM((96,), jnp.int32),
            pltpu.VMEM((96,), jnp.int32),
            pltpu.VMEM((96, D), jnp.float32),
            pltpu.VMEM((96, D), jnp.float32),
            pltpu.VMEM((96, D), jnp.float32),
            pltpu.VMEM((96, D), jnp.float32),
            pltpu.VMEM((96, D), jnp.float32),
            pltpu.VMEM((32,), jnp.int32),
            pltpu.VMEM((64, D), jnp.float32),
            pltpu.VMEM_SHARED((SLABR, D), jnp.float32),
            pltpu.VMEM_SHARED((SLABR, D), jnp.float32),
            pltpu.SemaphoreType.DMA,
        ],
    )
    def k(kt_h, q_h, vt_h, bs_h, bd_h, cn_h, acc_h, den_h,
          sv, dv, kidx, qidx, kst, qst, vst, stg_m, stg_d, cv, zb,
          slab_a, slab_d, sem):
        sid = lax.axis_index("s")
        c = lax.axis_index("c")
        iota = lax.iota(jnp.int32, 16)

        def take16(x, idx):
            dn = lax.GatherDimensionNumbers(offset_dims=(),
                                            collapsed_slice_dims=(0,),
                                            start_index_map=(0,))
            return lax.gather(x, idx[:, None], dn, (1,),
                              mode=lax.GatherScatterMode.PROMISE_IN_BOUNDS)

        z16 = jnp.zeros((16,), jnp.float32)

        def zrow(r, _):
            for l in range(8):
                zb[r, pl.ds(l * 16, 16)] = z16
            return _
        lax.fori_loop(0, 64, zrow, 0)

        for b in range(b_lo, b_lo + nbk):
            @pl.when((b % 2) == c)
            def _():
                def clr(kk, _):
                    pltpu.sync_copy(zb, slab_a.at[pl.ds(sid * 128 + kk * 64, 64)])
                    pltpu.sync_copy(zb, slab_d.at[pl.ds(sid * 128 + kk * 64, 64)])
                    return _
                lax.fori_loop(0, 2, clr, 0)
                plsc.subcore_barrier()

                def rloop(r, _):
                    reg = sid * 20 + r
                    pltpu.sync_copy(cn_h.at[pl.ds(reg * 32, 32)], cv)
                    if b < 16:
                        cvec = cv[pl.ds(0, 16)]
                        cnt = cvec[b]
                    else:
                        cvec = cv[pl.ds(16, 16)]
                        cnt = cvec[b - 16]
                    nck = (cnt + 95) // 96

                    def chunk(ck, _):
                        off = (reg * NBK + b) * CAP + ck * 96
                        pltpu.sync_copy(bs_h.at[pl.ds(off, 96)], sv)
                        pltpu.sync_copy(bd_h.at[pl.ds(off, 96)], dv)
                        rem = cnt - ck * 96

                        def grp(g, _):
                            sl = pl.ds(g * 16, 16)
                            s16 = sv[sl]
                            d16 = dv[sl]
                            valid = (iota + g * 16) < jnp.broadcast_to(rem, (16,))
                            kidx[sl] = jnp.where(valid, s16, 0)
                            qidx[sl] = jnp.where(valid, d16, 0)
                            dv[sl] = jnp.where(valid, d16 - b * BROW, DUMMY)
                            return _
                        lax.fori_loop(0, 6, grp, 0)

                        cp1 = pltpu.async_copy(kt_h.at[kidx], kst, sem)
                        cp2 = pltpu.async_copy(q_h.at[qidx], qst, sem)
                        cp3 = pltpu.async_copy(vt_h.at[kidx], vst, sem)
                        cp1.wait()
                        cp2.wait()
                        cp3.wait()

                        def edge(e, _):
                            for h in range(H):
                                hs = pl.ds(h * 16, 16)
                                ktr = kst[e, hs]
                                qr = qst[e, hs]
                                s = ktr * qr
                                for kk in (8, 4, 2, 1):
                                    s = s + take16(s, (iota + kk) & 15)
                                ex = jnp.exp(s)
                                stg_m[e, hs] = vst[e, hs] * ex
                                stg_d[e, hs] = ex
                            return _
                        lax.fori_loop(0, 96, edge, 0)

                        pltpu.sync_copy(stg_m, slab_a.at[dv], add=True)
                        pltpu.sync_copy(stg_d, slab_d.at[dv], add=True)
                        return _
                    lax.fori_loop(0, nck, chunk, 0)
                    return _
                lax.fori_loop(0, 20, rloop, 0)
                plsc.subcore_barrier()

                rows = pl.ds(sid * 128, 128)
                drows = pl.ds((b - b_lo) * BROW + sid * 128, 128)
                pltpu.sync_copy(slab_a.at[rows], acc_h.at[drows])
                pltpu.sync_copy(slab_d.at[rows], den_h.at[drows])
                plsc.subcore_barrier()

    return k(kt, qd, vt, bs, bd, cn)


def _sc_edges(src_w, dst_w, src_wb, dst_wb, ktw, qp, vtw, ktwb, qa, vtwb):
    zpad = jnp.zeros((PADE - E,), jnp.int32)
    spad = jnp.full((PADE - E,), NPAD_OUT, jnp.int32)
    srcw = jnp.concatenate([src_w, zpad])
    dstw = jnp.concatenate([dst_w, spad])
    srcwb = jnp.concatenate([src_wb, zpad])
    dstwb = jnp.concatenate([dst_wb, spad])

    bsw, bdw, cnw, bswb, bdwb, cnwb = _sc_prepass(srcw, dstw, srcwb, dstwb)

    nlo = 13
    accp0, denp0 = _sc_main(ktw, qp, vtw, bsw, bdw, cnw, 0, nlo)
    accp1, denp1 = _sc_main(ktw, qp, vtw, bsw, bdw, cnw, nlo, NBK - nlo)
    acca0, dena0 = _sc_main(ktwb, qa, vtwb, bswb, bdwb, cnwb, 0, nlo)
    acca1, dena1 = _sc_main(ktwb, qa, vtwb, bswb, bdwb, cnwb, nlo, NBK - nlo)

    accp = jnp.concatenate([accp0, accp1])[:N]
    denp = jnp.concatenate([denp0, denp1])[:N]
    acca = jnp.concatenate([acca0, acca1])[:N]
    dena = jnp.concatenate([dena0, dena1])[:N]
    return accp, denp, acca, dena


def _fold_weights(params):
    sqrt_dk = math.sqrt(DK)
    out = {}
    for e, srct, dstt in (("writes", "author", "paper"),
                          ("written_by", "paper", "author")):
        att = params["att_%s" % e] * (params["pri_%s" % e] / sqrt_dk)[:, None, None]
        wk = params["Wk_%s" % srct].reshape(D, H, DK)
        wkt = jnp.einsum("ihd,hdf->ihf", wk, att).reshape(D, D)
        bkt = jnp.einsum("hd,hdf->hf", params["bk_%s" % srct].reshape(H, DK),
                         att).reshape(-1)
        wv = params["Wv_%s" % srct].reshape(D, H, DK)
        wvt = jnp.einsum("ihd,hdf->ihf", wv, params["msg_%s" % e]).reshape(D, D)
        bvt = jnp.einsum("hd,hdf->hf", params["bv_%s" % srct].reshape(H, DK),
                         params["msg_%s" % e]).reshape(-1)
        out[e] = (wkt, bkt, wvt, bvt)
    return out


@jax.jit
def kernel(x_paper, x_author, edge_index_writes, edge_index_written_by, params):
    fw = _fold_weights(params)
    wkt_w, bkt_w, wvt_w, bvt_w = fw["writes"]
    wkt_wb, bkt_wb, wvt_wb, bvt_wb = fw["written_by"]

    wcat_p = jnp.concatenate([params["Wq_paper"], wkt_wb, wvt_wb], axis=1)
    bcat_p = jnp.concatenate([params["bq_paper"], bkt_wb, bvt_wb], axis=0)
    wcat_a = jnp.concatenate([params["Wq_author"], wkt_w, wvt_w], axis=1)
    bcat_a = jnp.concatenate([params["bq_author"], bkt_w, bvt_w], axis=0)

    q_p, kt_wb, vt_wb = _proj(x_paper, wcat_p, bcat_p)
    q_a, kt_w, vt_w = _proj(x_author, wcat_a, bcat_a)

    accp, denp, acca, dena = _sc_edges(
        edge_index_writes[0], edge_index_writes[1],
        edge_index_written_by[0], edge_index_written_by[1],
        kt_w, q_p, vt_w, kt_wb, q_a, vt_wb)

    out_p = _finalize(accp.reshape(N, D), denp.reshape(N, D), x_paper,
                      params["Wa_paper"], params["ba_paper"],
                      params["skip_paper"].reshape(1, 1))
    out_a = _finalize(acca.reshape(N, D), dena.reshape(N, D), x_author,
                      params["Wa_author"], params["ba_author"],
                      params["skip_author"].reshape(1, 1))
    return (out_p, out_a)
